# Initial kernel scaffold; baseline (speedup 1.0000x reference)
import functools

import jax
import jax.numpy as jnp
from jax import lax
from jax.experimental import pallas as pl
from jax.experimental.pallas import tpu as pltpu
from jax.experimental.pallas import tpu_sc as plsc

DIM_E = 128
DIM_R = 64
NC = 2
NS = 16
NW = NC * NS
CHUNK = 128


@functools.lru_cache(maxsize=None)
def _make_sc_gather(batch: int):
  assert batch % (NW * CHUNK) == 0
  bpw = batch // NW
  nchunk = bpw // CHUNK
  mesh = plsc.VectorSubcoreMesh(core_axis_name="c", subcore_axis_name="s")

  @functools.partial(
      pl.kernel,
      mesh=mesh,
      out_type=(
          jax.ShapeDtypeStruct((batch, DIM_E), jnp.float32),
          jax.ShapeDtypeStruct((batch, DIM_R), jnp.float32),
      ),
      scratch_types=[
          pltpu.VMEM((nchunk, CHUNK), jnp.int32),
          pltpu.VMEM((nchunk, CHUNK), jnp.int32),
          pltpu.VMEM((CHUNK, DIM_E), jnp.float32),
          pltpu.VMEM((CHUNK, DIM_R), jnp.float32),
          pltpu.SemaphoreType.DMA,
          pltpu.SemaphoreType.DMA,
      ],
  )
  def sc_gather(ent_t, rel_t, eidx_h, ridx_h, out_e, out_r,
                eidx_v, ridx_v, erows, rrows, esem, rsem):
    wid = lax.axis_index("s") * NC + lax.axis_index("c")
    row0 = wid * nchunk
    base = wid * bpw

    pltpu.sync_copy(eidx_h.at[pl.ds(row0, nchunk)], eidx_v)
    pltpu.sync_copy(ridx_h.at[pl.ds(row0, nchunk)], ridx_v)

    @pl.loop(0, nchunk)
    def _(j):
      he = pltpu.async_copy(ent_t.at[eidx_v.at[j]], erows, esem)
      hr = pltpu.async_copy(rel_t.at[ridx_v.at[j]], rrows, rsem)
      he.wait()
      pltpu.sync_copy(erows, out_e.at[pl.ds(base + j * CHUNK, CHUNK)])
      hr.wait()
      pltpu.sync_copy(rrows, out_r.at[pl.ds(base + j * CHUNK, CHUNK)])

  return sc_gather


def kernel(table_entities, table_relations, entities_idx, relations_idx):
  b, h = entities_idx.shape
  batch = b * h
  eidx = entities_idx.astype(jnp.int32).reshape(batch // CHUNK, CHUNK)
  ridx = relations_idx.astype(jnp.int32).reshape(batch // CHUNK, CHUNK)
  out_e, out_r = _make_sc_gather(batch)(
      table_entities, table_relations, eidx, ridx)
  return (out_e.reshape(b, h, DIM_E), out_r.reshape(b, h, DIM_R))


# speedup vs baseline: 4.3675x; 4.3675x over previous
import functools

import jax
import jax.numpy as jnp
from jax import lax
from jax.experimental import pallas as pl
from jax.experimental.pallas import tpu as pltpu
from jax.experimental.pallas import tpu_sc as plsc

DIM_E = 128
DIM_R = 64
NC = 2
NS = 16
NW = NC * NS
CHUNK = 128


@functools.lru_cache(maxsize=None)
def _make_gather(batch: int, dim: int, tc_tiling: bool):
  assert batch % (NW * CHUNK) == 0
  bpw = batch // NW
  nchunk = bpw // CHUNK
  mesh = plsc.VectorSubcoreMesh(core_axis_name="c", subcore_axis_name="s")
  params = None if tc_tiling else pltpu.CompilerParams(
      use_tc_tiling_on_sc=False)

  @functools.partial(
      pl.kernel,
      mesh=mesh,
      out_type=jax.ShapeDtypeStruct((batch, dim), jnp.float32),
      compiler_params=params,
      scratch_types=[
          pltpu.VMEM((nchunk, CHUNK), jnp.int32),
          pltpu.VMEM((CHUNK, dim), jnp.float32),
          pltpu.SemaphoreType.DMA,
      ],
  )
  def sc_gather(tab, idx_h, out, idx_v, rows, gsem):
    wid = lax.axis_index("s") * NC + lax.axis_index("c")
    base = wid * bpw

    pltpu.sync_copy(idx_h.at[wid], idx_v)

    @pl.loop(0, nchunk)
    def _(j):
      pltpu.async_copy(tab.at[idx_v.at[j]], rows, gsem).wait()
      pltpu.sync_copy(rows, out.at[pl.ds(base + j * CHUNK, CHUNK)])

  return sc_gather


def kernel(table_entities, table_relations, entities_idx, relations_idx):
  b, h = entities_idx.shape
  batch = b * h
  nchunk = batch // (NW * CHUNK)
  eidx = entities_idx.astype(jnp.int32).reshape(NW, nchunk, CHUNK)
  ridx = relations_idx.astype(jnp.int32).reshape(NW, nchunk, CHUNK)
  out_e = _make_gather(batch, DIM_E, True)(table_entities, eidx)
  out_r = _make_gather(batch, DIM_R, False)(table_relations, ridx)
  return (out_e.reshape(b, h, DIM_E), out_r.reshape(b, h, DIM_R))


# speedup vs baseline: 5.9573x; 1.3640x over previous
import functools

import jax
import jax.numpy as jnp
from jax import lax
from jax.experimental import pallas as pl
from jax.experimental.pallas import tpu as pltpu
from jax.experimental.pallas import tpu_sc as plsc

VOCAB_E = 100000
VOCAB_R = 1000
DIM_E = 128
DIM_R = 64
NC = 2
NS = 16
NW = NC * NS
HIST = 50


@functools.lru_cache(maxsize=None)
def _make_sc_kernel(b: int):
  assert b % (2 * NW) == 0
  bpw = b // NW
  npair = bpw // 2
  mesh = plsc.VectorSubcoreMesh(core_axis_name="c", subcore_axis_name="s")

  @functools.partial(
      pl.kernel,
      mesh=mesh,
      out_type=(
          jax.ShapeDtypeStruct((b, HIST, DIM_E), jnp.float32),
          jax.ShapeDtypeStruct((b, HIST, DIM_R), jnp.float32),
      ),
      scratch_types=[
          pltpu.VMEM((VOCAB_R * DIM_R,), jnp.float32),
          pltpu.VMEM((bpw, 128), jnp.int32),
          pltpu.VMEM((2, 2, HIST, DIM_E), jnp.float32),
          pltpu.VMEM((2, HIST, DIM_R), jnp.float32),
          pltpu.SemaphoreType.DMA,
          pltpu.SemaphoreType.DMA,
          pltpu.SemaphoreType.DMA,
          pltpu.SemaphoreType.DMA,
          pltpu.SemaphoreType.DMA,
          pltpu.SemaphoreType.DMA,
      ],
  )
  def sc_kernel(etab, rtab, idx_h, out_e, out_r,
                rtab_v, idx_v, erows, rrows,
                ge0, ge1, we0, we1, wr0, wr1):
    wid = lax.axis_index("s") * NC + lax.axis_index("c")
    base = wid * bpw
    ge = (ge0, ge1)
    we = (we0, we1)
    wr = (wr0, wr1)

    pltpu.sync_copy(rtab, rtab_v)
    pltpu.sync_copy(idx_h.at[wid], idx_v)

    def issue_gathers(p, h):
      bi = 2 * p
      pltpu.async_copy(etab.at[idx_v.at[bi, pl.ds(0, HIST)]],
                       erows.at[h, 0], ge[h])
      pltpu.async_copy(etab.at[idx_v.at[bi + 1, pl.ds(0, HIST)]],
                       erows.at[h, 1], ge[h])

    def drain_gathers(h):
      pltpu.make_async_copy(out_e.at[pl.ds(base, 2)], erows.at[h], ge[h]).wait()

    def fill_rel(bi, r):
      for g in range(4):
        iv = idx_v[bi, pl.ds(64 + 16 * g, 16)]
        for l in range(16 if g < 3 else 2):
          row = 16 * g + l
          off = iv[l] * DIM_R
          for q in range(0, DIM_R, 16):
            rrows[r, row, pl.ds(q, 16)] = rtab_v[pl.ds(off + q, 16)]

    def ent_wb(q, h):
      pltpu.async_copy(erows.at[h], out_e.at[pl.ds(base + 2 * q, 2)], we[h])

    def wait_ent_wb(q, h):
      pltpu.make_async_copy(erows.at[h], out_e.at[pl.ds(base + 2 * q, 2)],
                            we[h]).wait()

    def rel_wb(bi, r):
      pltpu.async_copy(rrows.at[r], out_r.at[base + bi], wr[r])

    def wait_rel_wb(bi, r):
      pltpu.make_async_copy(rrows.at[r], out_r.at[base + bi], wr[r]).wait()

    def stage_b(q, h):
      bi = 2 * q
      for r in (0, 1):
        @pl.when(bi + r >= 2)
        def _():
          wait_rel_wb(bi + r - 2, r)
        fill_rel(bi + r, r)
        rel_wb(bi + r, r)
      drain_gathers(h)
      ent_wb(q, h)

    @pl.loop(0, npair, step=2)
    def _(p0):
      for dp in (0, 1):
        p = p0 + dp
        h = dp

        @pl.when(p >= 2)
        def _():
          wait_ent_wb(p - 2, h)
        issue_gathers(p, h)

        @pl.when(p >= 1)
        def _():
          stage_b(p - 1, 1 - h)

    stage_b(npair - 1, 1)
    wait_ent_wb(npair - 2, 0)
    wait_ent_wb(npair - 1, 1)
    wait_rel_wb(bpw - 2, 0)
    wait_rel_wb(bpw - 1, 1)

  return sc_kernel


def kernel(table_entities, table_relations, entities_idx, relations_idx):
  b, h = entities_idx.shape
  pad = jnp.zeros((b, 64 - h), jnp.int32)
  packed = jnp.concatenate(
      [entities_idx.astype(jnp.int32), pad,
       relations_idx.astype(jnp.int32), pad], axis=1)
  packed = packed.reshape(NW, b // NW, 128)
  rtab = table_relations.reshape(VOCAB_R * DIM_R)
  out_e, out_r = _make_sc_kernel(b)(table_entities, rtab, packed)
  return (out_e, out_r)


# speedup vs baseline: 12.0274x; 2.0189x over previous
import functools

import jax
import jax.numpy as jnp
from jax import lax
from jax.experimental import pallas as pl
from jax.experimental.pallas import tpu as pltpu
from jax.experimental.pallas import tpu_sc as plsc

VOCAB_R = 1000
DIM_E = 128
DIM_R = 64
RSTRIDE = 65
NC = 2
NS = 16
NW = NC * NS
HIST = 50


@functools.lru_cache(maxsize=None)
def _make_sc_kernel(b: int, hist: int):
  bpw = b // NW
  assert b % NW == 0 and bpw % 16 == 0
  mesh = plsc.VectorSubcoreMesh(core_axis_name="c", subcore_axis_name="s")

  @functools.partial(
      pl.kernel,
      mesh=mesh,
      out_type=(
          jax.ShapeDtypeStruct((hist, b, DIM_E), jnp.float32),
          jax.ShapeDtypeStruct((hist, DIM_R, b), jnp.float32),
      ),
      compiler_params=pltpu.CompilerParams(needs_layout_passes=False),
      scratch_types=[
          pltpu.VMEM((VOCAB_R * RSTRIDE,), jnp.float32),
          pltpu.VMEM((hist, bpw), jnp.int32),
          pltpu.VMEM((hist, bpw), jnp.int32),
          pltpu.VMEM((2, bpw, DIM_E), jnp.float32),
          pltpu.VMEM((2, DIM_R, bpw), jnp.float32),
          pltpu.SemaphoreType.DMA,
          pltpu.SemaphoreType.DMA,
          pltpu.SemaphoreType.DMA,
          pltpu.SemaphoreType.DMA,
          pltpu.SemaphoreType.DMA,
          pltpu.SemaphoreType.DMA,
      ],
  )
  def sc_kernel(etab, rtab, eidx_h, ridx_h, out_e, out_r,
                rtab_v, eidx_v, ridx_v, erows, rrows,
                ge0, ge1, we0, we1, wr0, wr1):
    wid = lax.axis_index("s") * NC + lax.axis_index("c")
    base = wid * bpw
    ge = (ge0, ge1)
    we = (we0, we1)
    wr = (wr0, wr1)

    pltpu.sync_copy(rtab, rtab_v)
    pltpu.sync_copy(eidx_h.at[:, pl.ds(base, bpw)], eidx_v)
    pltpu.sync_copy(ridx_h.at[:, pl.ds(base, bpw)], ridx_v)

    def issue_gather(h, s):
      pltpu.async_copy(etab.at[eidx_v.at[h]], erows.at[s], ge[s])

    def wait_gather(h, s):
      pltpu.make_async_copy(etab.at[eidx_v.at[h]], erows.at[s], ge[s]).wait()

    def ent_wb(h, s):
      pltpu.async_copy(erows.at[s], out_e.at[h, pl.ds(base, bpw)], we[s])

    def wait_ent_wb(h, s):
      pltpu.make_async_copy(erows.at[s], out_e.at[h, pl.ds(base, bpw)],
                            we[s]).wait()

    def rel_wb(h, s):
      pltpu.async_copy(rrows.at[s], out_r.at[h, :, pl.ds(base, bpw)], wr[s])

    def wait_rel_wb(h, s):
      pltpu.make_async_copy(rrows.at[s], out_r.at[h, :, pl.ds(base, bpw)],
                            wr[s]).wait()

    def fill_rel(h, s):
      for g in range(bpw // 16):
        iv = ridx_v[h, pl.ds(16 * g, 16)] * RSTRIDE
        for c0 in range(0, DIM_R, 4):
          vals = [plsc.load_gather(rtab_v, [iv + (c0 + k)]) for k in range(4)]
          for k in range(4):
            rrows[s, c0 + k, pl.ds(16 * g, 16)] = vals[k]

    def stage_b(h, s):
      @pl.when(h >= 2)
      def _():
        wait_rel_wb(h - 2, s)
      fill_rel(h, s)
      rel_wb(h, s)
      wait_gather(h, s)
      ent_wb(h, s)

    @pl.loop(0, hist, step=2)
    def _(h0):
      for dp in (0, 1):
        h = h0 + dp
        s = dp

        @pl.when(h >= 2)
        def _():
          wait_ent_wb(h - 2, s)
        issue_gather(h, s)

        @pl.when(h >= 1)
        def _():
          stage_b(h - 1, 1 - s)

    stage_b(hist - 1, 1)
    wait_ent_wb(hist - 2, 0)
    wait_ent_wb(hist - 1, 1)
    wait_rel_wb(hist - 2, 0)
    wait_rel_wb(hist - 1, 1)

  return sc_kernel


def kernel(table_entities, table_relations, entities_idx, relations_idx):
  b, h = entities_idx.shape
  eidx = entities_idx.astype(jnp.int32).T
  ridx = relations_idx.astype(jnp.int32).T
  rtab = jnp.pad(table_relations, ((0, 0), (0, RSTRIDE - DIM_R))).reshape(VOCAB_R * RSTRIDE)
  out_e, out_r = _make_sc_kernel(b, h)(table_entities, rtab, eidx, ridx)
  return (out_e.transpose(1, 0, 2), out_r.transpose(2, 0, 1))


# speedup vs baseline: 13.3177x; 1.1073x over previous
import functools

import jax
import jax.numpy as jnp
from jax import lax
from jax.experimental import pallas as pl
from jax.experimental.pallas import tpu as pltpu
from jax.experimental.pallas import tpu_sc as plsc

VOCAB_R = 1000
DIM_E = 128
DIM_R = 64
RSTRIDE = 65
NC = 2
NS = 16
NW = NC * NS
HIST = 50


@functools.lru_cache(maxsize=None)
def _make_sc_kernel(b: int, hist: int):
  bpw = b // NW
  assert b % NW == 0 and bpw % 16 == 0
  mesh = plsc.VectorSubcoreMesh(core_axis_name="c", subcore_axis_name="s")

  @functools.partial(
      pl.kernel,
      mesh=mesh,
      out_type=(
          jax.ShapeDtypeStruct((hist, b, DIM_E), jnp.float32),
          jax.ShapeDtypeStruct((hist, DIM_R, b), jnp.float32),
      ),
      compiler_params=pltpu.CompilerParams(needs_layout_passes=False),
      scratch_types=[
          pltpu.VMEM((VOCAB_R * RSTRIDE,), jnp.float32),
          pltpu.VMEM((hist, bpw), jnp.int32),
          pltpu.VMEM((hist, bpw), jnp.int32),
          pltpu.VMEM((2, bpw, DIM_E), jnp.float32),
          pltpu.VMEM((2, DIM_R, bpw), jnp.float32),
          pltpu.SemaphoreType.DMA,
          pltpu.SemaphoreType.DMA,
          pltpu.SemaphoreType.DMA,
          pltpu.SemaphoreType.DMA,
          pltpu.SemaphoreType.DMA,
          pltpu.SemaphoreType.DMA,
      ],
  )
  def sc_kernel(etab, rtab, eidx_h, ridx_h, out_e, out_r,
                rtab_v, eidx_v, ridx_v, erows, rrows,
                ge0, ge1, we0, we1, wr0, wr1):
    wid = lax.axis_index("s") * NC + lax.axis_index("c")
    base = wid * bpw
    ge = (ge0, ge1)
    we = (we0, we1)
    wr = (wr0, wr1)

    pltpu.sync_copy(rtab, rtab_v)
    pltpu.sync_copy(eidx_h.at[:, pl.ds(base, bpw)], eidx_v)
    pltpu.sync_copy(ridx_h.at[:, pl.ds(base, bpw)], ridx_v)

    def issue_gather(h, s):
      pltpu.async_copy(etab.at[eidx_v.at[h]], erows.at[s], ge[s])

    def wait_gather(h, s):
      pltpu.make_async_copy(etab.at[eidx_v.at[h]], erows.at[s], ge[s]).wait()

    def ent_wb(h, s):
      pltpu.async_copy(erows.at[s], out_e.at[h, pl.ds(base, bpw)], we[s])

    def wait_ent_wb(h, s):
      pltpu.make_async_copy(erows.at[s], out_e.at[h, pl.ds(base, bpw)],
                            we[s]).wait()

    def rel_wb(h, s):
      pltpu.async_copy(rrows.at[s], out_r.at[h, :, pl.ds(base, bpw)], wr[s])

    def wait_rel_wb(h, s):
      pltpu.make_async_copy(rrows.at[s], out_r.at[h, :, pl.ds(base, bpw)],
                            wr[s]).wait()

    def fill_rel(h, s):
      for g in range(0, bpw // 16, 2):
        iv0 = ridx_v[h, pl.ds(16 * g, 16)] * RSTRIDE
        iv1 = ridx_v[h, pl.ds(16 * (g + 1), 16)] * RSTRIDE
        for c0 in range(0, DIM_R, 4):
          vals = (
              [plsc.load_gather(rtab_v, [iv0 + (c0 + k)]) for k in range(4)]
              + [plsc.load_gather(rtab_v, [iv1 + (c0 + k)]) for k in range(4)])
          for k in range(4):
            rrows[s, c0 + k, pl.ds(16 * g, 16)] = vals[k]
            rrows[s, c0 + k, pl.ds(16 * (g + 1), 16)] = vals[4 + k]

    def stage_b(h, s):
      @pl.when(h >= 2)
      def _():
        wait_rel_wb(h - 2, s)
      fill_rel(h, s)
      rel_wb(h, s)
      wait_gather(h, s)
      ent_wb(h, s)

    @pl.loop(0, hist, step=2)
    def _(h0):
      for dp in (0, 1):
        h = h0 + dp
        s = dp

        @pl.when(h >= 2)
        def _():
          wait_ent_wb(h - 2, s)
        issue_gather(h, s)

        @pl.when(h >= 1)
        def _():
          stage_b(h - 1, 1 - s)

    stage_b(hist - 1, 1)
    wait_ent_wb(hist - 2, 0)
    wait_ent_wb(hist - 1, 1)
    wait_rel_wb(hist - 2, 0)
    wait_rel_wb(hist - 1, 1)

  return sc_kernel


def kernel(table_entities, table_relations, entities_idx, relations_idx):
  b, h = entities_idx.shape
  eidx = entities_idx.astype(jnp.int32).T
  ridx = relations_idx.astype(jnp.int32).T
  rtab = jnp.pad(table_relations, ((0, 0), (0, RSTRIDE - DIM_R))).reshape(VOCAB_R * RSTRIDE)
  out_e, out_r = _make_sc_kernel(b, h)(table_entities, rtab, eidx, ridx)
  return (out_e.transpose(1, 0, 2), out_r.transpose(2, 0, 1))


# speedup vs baseline: 13.7362x; 1.0314x over previous
import functools

import jax
import jax.numpy as jnp
from jax import lax
from jax.experimental import pallas as pl
from jax.experimental.pallas import tpu as pltpu
from jax.experimental.pallas import tpu_sc as plsc

VOCAB_R = 1000
DIM_E = 128
DIM_R = 64
RSTRIDE = 65
NC = 2
NS = 16
NW = NC * NS
HIST = 50


@functools.lru_cache(maxsize=None)
def _make_sc_kernel(b: int, hist: int):
  bpw = b // NW
  assert b % NW == 0 and bpw % 16 == 0
  mesh = plsc.VectorSubcoreMesh(core_axis_name="c", subcore_axis_name="s")

  @functools.partial(
      pl.kernel,
      mesh=mesh,
      out_type=(
          jax.ShapeDtypeStruct((hist, b, DIM_E), jnp.float32),
          jax.ShapeDtypeStruct((hist, DIM_R, b), jnp.float32),
      ),
      compiler_params=pltpu.CompilerParams(needs_layout_passes=False),
      scratch_types=[
          pltpu.VMEM((VOCAB_R * RSTRIDE,), jnp.float32),
          pltpu.VMEM((hist, bpw), jnp.int32),
          pltpu.VMEM((hist, bpw), jnp.int32),
          pltpu.VMEM((2, bpw, DIM_E), jnp.float32),
          pltpu.VMEM((2, DIM_R, bpw), jnp.float32),
          pltpu.SemaphoreType.DMA,
          pltpu.SemaphoreType.DMA,
          pltpu.SemaphoreType.DMA,
          pltpu.SemaphoreType.DMA,
          pltpu.SemaphoreType.DMA,
          pltpu.SemaphoreType.DMA,
      ],
  )
  def sc_kernel(etab, rtab, eidx_h, ridx_h, out_e, out_r,
                rtab_v, eidx_v, ridx_v, erows, rrows,
                ge0, ge1, we0, we1, wr0, wr1):
    wid = lax.axis_index("s") * NC + lax.axis_index("c")
    base = wid * bpw
    ge = (ge0, ge1)
    we = (we0, we1)
    wr = (wr0, wr1)

    pltpu.sync_copy(rtab, rtab_v)
    pltpu.sync_copy(eidx_h.at[:, pl.ds(base, bpw)], eidx_v)
    pltpu.sync_copy(ridx_h.at[:, pl.ds(base, bpw)], ridx_v)

    def issue_gather(h, s):
      pltpu.async_copy(etab.at[eidx_v.at[h]], erows.at[s], ge[s])

    def wait_gather(h, s):
      pltpu.make_async_copy(etab.at[eidx_v.at[h]], erows.at[s], ge[s]).wait()

    def ent_wb(h, s):
      pltpu.async_copy(erows.at[s], out_e.at[h, pl.ds(base, bpw)], we[s])

    def wait_ent_wb(h, s):
      pltpu.make_async_copy(erows.at[s], out_e.at[h, pl.ds(base, bpw)],
                            we[s]).wait()

    def rel_wb(h, s):
      pltpu.async_copy(rrows.at[s], out_r.at[h, :, pl.ds(base, bpw)], wr[s])

    def wait_rel_wb(h, s):
      pltpu.make_async_copy(rrows.at[s], out_r.at[h, :, pl.ds(base, bpw)],
                            wr[s]).wait()

    def fill_rel(h, s):
      for g in range(0, bpw // 16, 4):
        ivs = [ridx_v[h, pl.ds(16 * (g + j), 16)] * RSTRIDE for j in range(4)]
        for c0 in range(0, DIM_R, 4):
          vals = [plsc.load_gather(rtab_v, [ivs[j] + (c0 + k)])
                  for j in range(4) for k in range(4)]
          for j in range(4):
            for k in range(4):
              rrows[s, c0 + k, pl.ds(16 * (g + j), 16)] = vals[4 * j + k]

    def stage_b(h, s):
      @pl.when(h >= 2)
      def _():
        wait_rel_wb(h - 2, s)
      fill_rel(h, s)
      rel_wb(h, s)
      wait_gather(h, s)
      ent_wb(h, s)

    @pl.loop(0, hist, step=2)
    def _(h0):
      for dp in (0, 1):
        h = h0 + dp
        s = dp

        @pl.when(h >= 2)
        def _():
          wait_ent_wb(h - 2, s)
        issue_gather(h, s)

        @pl.when(h >= 1)
        def _():
          stage_b(h - 1, 1 - s)

    stage_b(hist - 1, 1)
    wait_ent_wb(hist - 2, 0)
    wait_ent_wb(hist - 1, 1)
    wait_rel_wb(hist - 2, 0)
    wait_rel_wb(hist - 1, 1)

  return sc_kernel


def kernel(table_entities, table_relations, entities_idx, relations_idx):
  b, h = entities_idx.shape
  eidx = entities_idx.astype(jnp.int32).T
  ridx = relations_idx.astype(jnp.int32).T
  rtab = jnp.pad(table_relations, ((0, 0), (0, RSTRIDE - DIM_R))).reshape(VOCAB_R * RSTRIDE)
  out_e, out_r = _make_sc_kernel(b, h)(table_entities, rtab, eidx, ridx)
  return (out_e.transpose(1, 0, 2), out_r.transpose(2, 0, 1))
